# Initial kernel scaffold; baseline (speedup 1.0000x reference)
import jax
import jax.numpy as jnp
from jax.experimental import pallas as pl


def kernel(x0, edge_index0, x1, edge_index1, W0_0, b0_0, W0_1, b0_1, W0_2, b0_2, W1_0, b1_0, W1_1, b1_1, W1_2, b1_2):
    raise NotImplementedError("write your pallas kernel here")


# speedup vs baseline: 15.4483x; 15.4483x over previous
import functools

import jax
import jax.numpy as jnp
from jax import lax
from jax.experimental import pallas as pl
from jax.experimental.pallas import tpu as pltpu
from jax.experimental.pallas import tpu_sc as plsc

N_NODES = 10000
N_EDGES = 320000
D = 128
G = 2

CHUNK = 128
NCH = N_EDGES // CHUNK
NSUB = 16
CHT = 160
CHT_LAST = NCH - 15 * CHT
ROWS_T = 624
ROWS_LAST = N_NODES - 15 * ROWS_T
KB = 8
NB = CHT // KB
NB_LAST = 12
TAIL = CHT_LAST - NB_LAST * KB

_sc_mesh = plsc.VectorSubcoreMesh(core_axis_name="c", subcore_axis_name="s")


DEG_R = 80


@functools.partial(
    pl.kernel,
    out_type=jax.ShapeDtypeStruct((G, DEG_R, CHUNK), jnp.float32),
    mesh=_sc_mesh,
    compiler_params=pltpu.CompilerParams(needs_layout_passes=False),
    scratch_types=[
        pltpu.VMEM((CHT, CHUNK), jnp.int32),
        pltpu.VMEM((DEG_R, CHUNK), jnp.float32),
        pltpu.VMEM((DEG_R * CHUNK,), jnp.float32),
        pltpu.VMEM((DEG_R,), jnp.int32),
        pltpu.VMEM_SHARED((DEG_R, CHUNK), jnp.float32),
    ],
)
def _sc_degree(ei_hbm, deg_hbm, dst_v, deg_v, deg_f, rows_i, acc_sh):
    c = lax.axis_index("c")
    s = lax.axis_index("s")

    def zfill(i, _):
        for jj in range(CHUNK // 16):
            deg_v[i, pl.ds(jj * 16, 16)] = jnp.zeros((16,), jnp.float32)
        return ()

    lax.fori_loop(0, DEG_R, zfill, ())
    for k in range(DEG_R // 16):
        rows_i[pl.ds(k * 16, 16)] = lax.iota(jnp.int32, 16) + (k * 16)

    @pl.when(s == 0)
    def _():
        pltpu.sync_copy(deg_v, acc_sh)

    @pl.when(s < NSUB - 1)
    def _():
        pltpu.sync_copy(ei_hbm.at[2 * c + 1, pl.ds(s * CHT, CHT)], dst_v)

    @pl.when(s == NSUB - 1)
    def _():
        pltpu.sync_copy(ei_hbm.at[2 * c + 1, pl.ds(15 * CHT, CHT_LAST)],
                        dst_v.at[pl.ds(0, CHT_LAST)])

    plsc.subcore_barrier()
    ones16 = jnp.full((16,), 1.0, jnp.float32)

    def zfill2(i, _):
        for jj in range(CHUNK // 16):
            deg_f[pl.ds(i * CHUNK + jj * 16, 16)] = jnp.zeros((16,), jnp.float32)
        return ()

    lax.fori_loop(0, DEG_R, zfill2, ())

    def body(j, _):
        for k in range(CHUNK // 16):
            idx = dst_v[j, pl.ds(k * 16, 16)]
            plsc.addupdate_scatter(deg_f, [idx], ones16)
        return ()

    lax.fori_loop(0, CHT_LAST, body, ())

    @pl.when(s < NSUB - 1)
    def _():
        lax.fori_loop(CHT_LAST, CHT, body, ())

    def tile_rows(r, _):
        for k in range(CHUNK // 16):
            deg_v[r, pl.ds(k * 16, 16)] = deg_f[pl.ds(r * CHUNK + k * 16, 16)]
        return ()

    lax.fori_loop(0, DEG_R, tile_rows, ())
    pltpu.sync_copy(deg_v, acc_sh.at[rows_i], add=True)
    plsc.subcore_barrier()

    @pl.when(s == 0)
    def _():
        pltpu.sync_copy(acc_sh, deg_hbm.at[c])


@functools.partial(
    pl.kernel,
    out_type=jax.ShapeDtypeStruct((G, N_NODES, D), jnp.float32),
    mesh=_sc_mesh,
    scratch_types=[
        pltpu.VMEM((KB, CHUNK), jnp.int32),
        pltpu.VMEM((KB, CHUNK), jnp.int32),
        pltpu.VMEM((CHUNK, D), jnp.float32),
        pltpu.VMEM_SHARED((N_NODES, D), jnp.float32),
        pltpu.SemaphoreType.DMA,
    ],
)
def _sc_aggregate(hp_hbm, ei_hbm, acc_hbm, src_v, dst_v, rows_v, acc_sh, sem):
    c = lax.axis_index("c")
    s = lax.axis_index("s")
    row0 = s * ROWS_T
    chunk0 = s * CHT

    @pl.when(s < NSUB - 1)
    def _():
        pltpu.sync_copy(hp_hbm.at[pl.ds(c * N_NODES + row0, ROWS_T)],
                        acc_sh.at[pl.ds(row0, ROWS_T)])

    @pl.when(s == NSUB - 1)
    def _():
        pltpu.sync_copy(hp_hbm.at[pl.ds(c * N_NODES + 15 * ROWS_T, ROWS_LAST)],
                        acc_sh.at[pl.ds(15 * ROWS_T, ROWS_LAST)])

    plsc.subcore_barrier()

    coff = c * N_NODES

    def chunk_step(j):
        pltpu.async_copy(hp_hbm.at[src_v.at[j]], rows_v, sem).wait()
        pltpu.sync_copy(rows_v, acc_sh.at[dst_v.at[j]], add=True)

    def batch(b, _):
        off = chunk0 + b * KB
        pltpu.sync_copy(ei_hbm.at[2 * c, pl.ds(off, KB)], src_v)
        pltpu.sync_copy(ei_hbm.at[2 * c + 1, pl.ds(off, KB)], dst_v)

        def add_off(i, _):
            for jj in range(CHUNK // 16):
                sl = pl.ds(jj * 16, 16)
                src_v[i, sl] = src_v[i, sl] + coff
            return ()

        lax.fori_loop(0, KB, add_off, ())
        for j in range(KB):
            chunk_step(j)
        return ()

    lax.fori_loop(0, NB_LAST, batch, ())

    @pl.when(s < NSUB - 1)
    def _():
        lax.fori_loop(NB_LAST, NB, batch, ())

    @pl.when(s == NSUB - 1)
    def _():
        off = 15 * CHT + NB_LAST * KB
        pltpu.sync_copy(ei_hbm.at[2 * c, pl.ds(off, TAIL)],
                        src_v.at[pl.ds(0, TAIL)])
        pltpu.sync_copy(ei_hbm.at[2 * c + 1, pl.ds(off, TAIL)],
                        dst_v.at[pl.ds(0, TAIL)])

        def add_off(i, _):
            for jj in range(CHUNK // 16):
                sl = pl.ds(jj * 16, 16)
                src_v[i, sl] = src_v[i, sl] + coff
            return ()

        lax.fori_loop(0, TAIL, add_off, ())
        for j in range(TAIL):
            chunk_step(j)

    plsc.subcore_barrier()

    @pl.when(s < NSUB - 1)
    def _():
        pltpu.sync_copy(acc_sh.at[pl.ds(row0, ROWS_T)],
                        acc_hbm.at[c, pl.ds(row0, ROWS_T)])

    @pl.when(s == NSUB - 1)
    def _():
        pltpu.sync_copy(acc_sh.at[pl.ds(15 * ROWS_T, ROWS_LAST)],
                        acc_hbm.at[c, pl.ds(15 * ROWS_T, ROWS_LAST)])


def _dinv(deg_ref):
    return lax.rsqrt(deg_ref[0] + 1.0)


def _tc_first_body(deg_ref, x_ref, w_ref, hp_ref):
    h = jnp.dot(x_ref[0], w_ref[0], preferred_element_type=jnp.float32,
                precision=lax.Precision.HIGHEST)
    hp_ref[0] = h * _dinv(deg_ref)


def _tc_mid_body(deg_ref, acc_ref, b_ref, w_ref, hp_ref):
    dinv = _dinv(deg_ref)
    pre = acc_ref[0] * dinv + b_ref[0]
    h = jnp.dot(pre, w_ref[0], preferred_element_type=jnp.float32,
                precision=lax.Precision.HIGHEST)
    hp_ref[0] = h * dinv


def _tc_last_body(deg_ref, acc_ref, b_ref, out_ref):
    out_ref[0] = acc_ref[0] * _dinv(deg_ref) + b_ref[0]


def _bspec(shape):
    return pl.BlockSpec((1,) + shape, lambda g: (g,) + (0,) * len(shape))


_deg_spec = _bspec((N_NODES, D))
_nd_spec = _bspec((N_NODES, D))
_w_spec = _bspec((D, D))
_b_spec = _bspec((1, D))
_nd_out = jax.ShapeDtypeStruct((G, N_NODES, D), jnp.float32)

_tc_first = pl.pallas_call(
    _tc_first_body, grid=(G,),
    in_specs=[_deg_spec, _nd_spec, _w_spec],
    out_specs=_nd_spec, out_shape=_nd_out)

_tc_mid = pl.pallas_call(
    _tc_mid_body, grid=(G,),
    in_specs=[_deg_spec, _nd_spec, _b_spec, _w_spec],
    out_specs=_nd_spec, out_shape=_nd_out)

_tc_last = pl.pallas_call(
    _tc_last_body, grid=(G,),
    in_specs=[_deg_spec, _nd_spec, _b_spec],
    out_specs=_nd_spec, out_shape=_nd_out)


def kernel(x0, edge_index0, x1, edge_index1,
           W0_0, b0_0, W0_1, b0_1, W0_2, b0_2,
           W1_0, b1_0, W1_1, b1_1, W1_2, b1_2):
    ei = jnp.stack([edge_index0.astype(jnp.int32),
                    edge_index1.astype(jnp.int32)])
    eiR = ei.reshape(2 * G, NCH, CHUNK)
    X = jnp.stack([x0, x1])
    Ws = [jnp.stack([W0_0, W1_0]), jnp.stack([W0_1, W1_1]),
          jnp.stack([W0_2, W1_2])]
    Bs = [jnp.stack([b0_0, b1_0]).reshape(G, 1, D),
          jnp.stack([b0_1, b1_1]).reshape(G, 1, D),
          jnp.stack([b0_2, b1_2]).reshape(G, 1, D)]

    deg_flat = _sc_degree(eiR).reshape(G, DEG_R * CHUNK)[:, :N_NODES]
    deg16 = jnp.broadcast_to(deg_flat[:, :, None], (G, N_NODES, D))
    hp = _tc_first(deg16, X, Ws[0])
    acc = _sc_aggregate(hp.reshape(G * N_NODES, D), eiR)
    hp = _tc_mid(deg16, acc, Bs[0], Ws[1])
    acc = _sc_aggregate(hp.reshape(G * N_NODES, D), eiR)
    hp = _tc_mid(deg16, acc, Bs[1], Ws[2])
    acc = _sc_aggregate(hp.reshape(G * N_NODES, D), eiR)
    out = _tc_last(deg16, acc, Bs[2])
    return out.reshape(G * N_NODES, D)


# speedup vs baseline: 21.6691x; 1.4027x over previous
import functools

import jax
import jax.numpy as jnp
from jax import lax
from jax.experimental import pallas as pl
from jax.experimental.pallas import tpu as pltpu
from jax.experimental.pallas import tpu_sc as plsc

N_NODES = 10000
N_EDGES = 320000
D = 128
G = 2

CHUNK = 128
NCH = N_EDGES // CHUNK
NSUB = 16
CHT = 160
CHT_LAST = NCH - 15 * CHT
ROWS_T = 624
ROWS_LAST = N_NODES - 15 * ROWS_T
KB = 8
NB = CHT // KB
NB_LAST = 12
TAIL = CHT_LAST - NB_LAST * KB

_sc_mesh = plsc.VectorSubcoreMesh(core_axis_name="c", subcore_axis_name="s")


DEG_R = 80


@functools.partial(
    pl.kernel,
    out_type=jax.ShapeDtypeStruct((G, DEG_R, CHUNK), jnp.float32),
    mesh=_sc_mesh,
    compiler_params=pltpu.CompilerParams(needs_layout_passes=False),
    scratch_types=[
        pltpu.VMEM((CHT, CHUNK), jnp.int32),
        pltpu.VMEM((DEG_R, CHUNK), jnp.float32),
        pltpu.VMEM((DEG_R * CHUNK,), jnp.float32),
        pltpu.VMEM((DEG_R,), jnp.int32),
        pltpu.VMEM_SHARED((DEG_R, CHUNK), jnp.float32),
    ],
)
def _sc_degree(ei_hbm, deg_hbm, dst_v, deg_v, deg_f, rows_i, acc_sh):
    c = lax.axis_index("c")
    s = lax.axis_index("s")

    def zfill(i, _):
        for jj in range(CHUNK // 16):
            deg_v[i, pl.ds(jj * 16, 16)] = jnp.zeros((16,), jnp.float32)
        return ()

    lax.fori_loop(0, DEG_R, zfill, ())
    for k in range(DEG_R // 16):
        rows_i[pl.ds(k * 16, 16)] = lax.iota(jnp.int32, 16) + (k * 16)

    @pl.when(s == 0)
    def _():
        pltpu.sync_copy(deg_v, acc_sh)

    @pl.when(s < NSUB - 1)
    def _():
        pltpu.sync_copy(ei_hbm.at[2 * c + 1, pl.ds(s * CHT, CHT)], dst_v)

    @pl.when(s == NSUB - 1)
    def _():
        pltpu.sync_copy(ei_hbm.at[2 * c + 1, pl.ds(15 * CHT, CHT_LAST)],
                        dst_v.at[pl.ds(0, CHT_LAST)])

    plsc.subcore_barrier()
    ones16 = jnp.full((16,), 1.0, jnp.float32)

    def zfill2(i, _):
        for jj in range(CHUNK // 16):
            deg_f[pl.ds(i * CHUNK + jj * 16, 16)] = jnp.zeros((16,), jnp.float32)
        return ()

    lax.fori_loop(0, DEG_R, zfill2, ())

    def body(j, _):
        for k in range(CHUNK // 16):
            idx = dst_v[j, pl.ds(k * 16, 16)]
            plsc.addupdate_scatter(deg_f, [idx], ones16)
        return ()

    lax.fori_loop(0, CHT_LAST, body, ())

    @pl.when(s < NSUB - 1)
    def _():
        lax.fori_loop(CHT_LAST, CHT, body, ())

    def tile_rows(r, _):
        for k in range(CHUNK // 16):
            deg_v[r, pl.ds(k * 16, 16)] = deg_f[pl.ds(r * CHUNK + k * 16, 16)]
        return ()

    lax.fori_loop(0, DEG_R, tile_rows, ())
    pltpu.sync_copy(deg_v, acc_sh.at[rows_i], add=True)
    plsc.subcore_barrier()

    @pl.when(s == 0)
    def _():
        pltpu.sync_copy(acc_sh, deg_hbm.at[c])


@functools.partial(
    pl.kernel,
    out_type=jax.ShapeDtypeStruct((G, N_NODES, D), jnp.float32),
    mesh=_sc_mesh,
    scratch_types=[
        pltpu.VMEM((KB, CHUNK), jnp.int32),
        pltpu.VMEM((KB, CHUNK), jnp.int32),
        pltpu.VMEM((CHUNK, D), jnp.float32),
        pltpu.VMEM((CHUNK, D), jnp.float32),
        pltpu.VMEM_SHARED((N_NODES, D), jnp.float32),
        pltpu.SemaphoreType.DMA,
        pltpu.SemaphoreType.DMA,
        pltpu.SemaphoreType.DMA,
        pltpu.SemaphoreType.DMA,
    ],
)
def _sc_aggregate(hp_hbm, ei_hbm, acc_hbm, src_v, dst_v, rows_a, rows_b,
                  acc_sh, gsa, gsb, ssa, ssb):
    c = lax.axis_index("c")
    s = lax.axis_index("s")
    row0 = s * ROWS_T
    chunk0 = s * CHT

    @pl.when(s < NSUB - 1)
    def _():
        pltpu.sync_copy(hp_hbm.at[pl.ds(c * N_NODES + row0, ROWS_T)],
                        acc_sh.at[pl.ds(row0, ROWS_T)])

    @pl.when(s == NSUB - 1)
    def _():
        pltpu.sync_copy(hp_hbm.at[pl.ds(c * N_NODES + 15 * ROWS_T, ROWS_LAST)],
                        acc_sh.at[pl.ds(15 * ROWS_T, ROWS_LAST)])

    plsc.subcore_barrier()

    coff = c * N_NODES

    bufs = [(rows_a, gsa, ssa), (rows_b, gsb, ssb)]

    def batch(b, _):
        off = chunk0 + b * KB
        pltpu.sync_copy(ei_hbm.at[2 * c, pl.ds(off, KB)], src_v)
        pltpu.sync_copy(ei_hbm.at[2 * c + 1, pl.ds(off, KB)], dst_v)

        def add_off(i, _):
            for jj in range(CHUNK // 16):
                sl = pl.ds(jj * 16, 16)
                src_v[i, sl] = src_v[i, sl] + coff
            return ()

        lax.fori_loop(0, KB, add_off, ())
        gathers = [None, None]
        for p in range(2):
            rows, gs, _ = bufs[p]
            gathers[p] = pltpu.async_copy(hp_hbm.at[src_v.at[p]], rows, gs)
        for j in range(KB):
            rows, gs, ss = bufs[j % 2]
            gathers[j % 2].wait()
            pltpu.async_copy(rows, acc_sh.at[dst_v.at[j]], ss,
                             add=True).wait()
            if j + 2 < KB:
                gathers[j % 2] = pltpu.async_copy(
                    hp_hbm.at[src_v.at[j + 2]], rows, gs)
        return ()

    lax.fori_loop(0, NB_LAST, batch, ())

    @pl.when(s < NSUB - 1)
    def _():
        lax.fori_loop(NB_LAST, NB, batch, ())

    @pl.when(s == NSUB - 1)
    def _():
        off = 15 * CHT + NB_LAST * KB
        pltpu.sync_copy(ei_hbm.at[2 * c, pl.ds(off, TAIL)],
                        src_v.at[pl.ds(0, TAIL)])
        pltpu.sync_copy(ei_hbm.at[2 * c + 1, pl.ds(off, TAIL)],
                        dst_v.at[pl.ds(0, TAIL)])

        def add_off(i, _):
            for jj in range(CHUNK // 16):
                sl = pl.ds(jj * 16, 16)
                src_v[i, sl] = src_v[i, sl] + coff
            return ()

        lax.fori_loop(0, TAIL, add_off, ())
        for j in range(TAIL):
            pltpu.async_copy(hp_hbm.at[src_v.at[j]], rows_a, gsa).wait()
            pltpu.async_copy(rows_a, acc_sh.at[dst_v.at[j]], ssa,
                             add=True).wait()

    plsc.subcore_barrier()

    @pl.when(s < NSUB - 1)
    def _():
        pltpu.sync_copy(acc_sh.at[pl.ds(row0, ROWS_T)],
                        acc_hbm.at[c, pl.ds(row0, ROWS_T)])

    @pl.when(s == NSUB - 1)
    def _():
        pltpu.sync_copy(acc_sh.at[pl.ds(15 * ROWS_T, ROWS_LAST)],
                        acc_hbm.at[c, pl.ds(15 * ROWS_T, ROWS_LAST)])


def _dinv(deg_ref):
    return lax.rsqrt(deg_ref[0] + 1.0)


def _tc_first_body(deg_ref, x_ref, w_ref, hp_ref):
    h = jnp.dot(x_ref[0], w_ref[0], preferred_element_type=jnp.float32,
                precision=lax.Precision.HIGHEST)
    hp_ref[0] = h * _dinv(deg_ref)


def _tc_mid_body(deg_ref, acc_ref, b_ref, w_ref, hp_ref):
    dinv = _dinv(deg_ref)
    pre = acc_ref[0] * dinv + b_ref[0]
    h = jnp.dot(pre, w_ref[0], preferred_element_type=jnp.float32,
                precision=lax.Precision.HIGHEST)
    hp_ref[0] = h * dinv


def _tc_last_body(deg_ref, acc_ref, b_ref, out_ref):
    out_ref[0] = acc_ref[0] * _dinv(deg_ref) + b_ref[0]


def _bspec(shape):
    return pl.BlockSpec((1,) + shape, lambda g: (g,) + (0,) * len(shape))


_deg_spec = _bspec((N_NODES, D))
_nd_spec = _bspec((N_NODES, D))
_w_spec = _bspec((D, D))
_b_spec = _bspec((1, D))
_nd_out = jax.ShapeDtypeStruct((G, N_NODES, D), jnp.float32)

_tc_first = pl.pallas_call(
    _tc_first_body, grid=(G,),
    in_specs=[_deg_spec, _nd_spec, _w_spec],
    out_specs=_nd_spec, out_shape=_nd_out)

_tc_mid = pl.pallas_call(
    _tc_mid_body, grid=(G,),
    in_specs=[_deg_spec, _nd_spec, _b_spec, _w_spec],
    out_specs=_nd_spec, out_shape=_nd_out)

_tc_last = pl.pallas_call(
    _tc_last_body, grid=(G,),
    in_specs=[_deg_spec, _nd_spec, _b_spec],
    out_specs=_nd_spec, out_shape=_nd_out)


def kernel(x0, edge_index0, x1, edge_index1,
           W0_0, b0_0, W0_1, b0_1, W0_2, b0_2,
           W1_0, b1_0, W1_1, b1_1, W1_2, b1_2):
    ei = jnp.stack([edge_index0.astype(jnp.int32),
                    edge_index1.astype(jnp.int32)])
    eiR = ei.reshape(2 * G, NCH, CHUNK)
    X = jnp.stack([x0, x1])
    Ws = [jnp.stack([W0_0, W1_0]), jnp.stack([W0_1, W1_1]),
          jnp.stack([W0_2, W1_2])]
    Bs = [jnp.stack([b0_0, b1_0]).reshape(G, 1, D),
          jnp.stack([b0_1, b1_1]).reshape(G, 1, D),
          jnp.stack([b0_2, b1_2]).reshape(G, 1, D)]

    deg_flat = _sc_degree(eiR).reshape(G, DEG_R * CHUNK)[:, :N_NODES]
    deg16 = jnp.broadcast_to(deg_flat[:, :, None], (G, N_NODES, D))
    hp = _tc_first(deg16, X, Ws[0])
    acc = _sc_aggregate(hp.reshape(G * N_NODES, D), eiR)
    hp = _tc_mid(deg16, acc, Bs[0], Ws[1])
    acc = _sc_aggregate(hp.reshape(G * N_NODES, D), eiR)
    hp = _tc_mid(deg16, acc, Bs[1], Ws[2])
    acc = _sc_aggregate(hp.reshape(G * N_NODES, D), eiR)
    out = _tc_last(deg16, acc, Bs[2])
    return out.reshape(G * N_NODES, D)


# speedup vs baseline: 25.6351x; 1.1830x over previous
import functools

import jax
import jax.numpy as jnp
from jax import lax
from jax.experimental import pallas as pl
from jax.experimental.pallas import tpu as pltpu
from jax.experimental.pallas import tpu_sc as plsc

N_NODES = 10000
N_EDGES = 320000
D = 128
G = 2

CHUNK = 128
NCH = N_EDGES // CHUNK
NSUB = 16
CHT = 160
CHT_LAST = NCH - 15 * CHT
ROWS_T = 624
ROWS_LAST = N_NODES - 15 * ROWS_T
KB = 8
NB = CHT // KB
NB_LAST = 12
TAIL = CHT_LAST - NB_LAST * KB

_sc_mesh = plsc.VectorSubcoreMesh(core_axis_name="c", subcore_axis_name="s")


DEG_R = 80


@functools.partial(
    pl.kernel,
    out_type=jax.ShapeDtypeStruct((G, DEG_R, CHUNK), jnp.float32),
    mesh=_sc_mesh,
    compiler_params=pltpu.CompilerParams(needs_layout_passes=False),
    scratch_types=[
        pltpu.VMEM((CHT, CHUNK), jnp.int32),
        pltpu.VMEM((DEG_R, CHUNK), jnp.float32),
        pltpu.VMEM((DEG_R * CHUNK,), jnp.float32),
        pltpu.VMEM((DEG_R,), jnp.int32),
        pltpu.VMEM_SHARED((DEG_R, CHUNK), jnp.float32),
    ],
)
def _sc_degree(ei_hbm, deg_hbm, dst_v, deg_v, deg_f, rows_i, acc_sh):
    c = lax.axis_index("c")
    s = lax.axis_index("s")

    def zfill(i, _):
        for jj in range(CHUNK // 16):
            deg_v[i, pl.ds(jj * 16, 16)] = jnp.zeros((16,), jnp.float32)
        return ()

    lax.fori_loop(0, DEG_R, zfill, ())
    for k in range(DEG_R // 16):
        rows_i[pl.ds(k * 16, 16)] = lax.iota(jnp.int32, 16) + (k * 16)

    @pl.when(s == 0)
    def _():
        pltpu.sync_copy(deg_v, acc_sh)

    @pl.when(s < NSUB - 1)
    def _():
        pltpu.sync_copy(ei_hbm.at[2 * c + 1, pl.ds(s * CHT, CHT)], dst_v)

    @pl.when(s == NSUB - 1)
    def _():
        pltpu.sync_copy(ei_hbm.at[2 * c + 1, pl.ds(15 * CHT, CHT_LAST)],
                        dst_v.at[pl.ds(0, CHT_LAST)])

    plsc.subcore_barrier()
    ones16 = jnp.full((16,), 1.0, jnp.float32)

    def zfill2(i, _):
        for jj in range(CHUNK // 16):
            deg_f[pl.ds(i * CHUNK + jj * 16, 16)] = jnp.zeros((16,), jnp.float32)
        return ()

    lax.fori_loop(0, DEG_R, zfill2, ())

    def body(j, _):
        for k in range(CHUNK // 16):
            idx = dst_v[j, pl.ds(k * 16, 16)]
            plsc.addupdate_scatter(deg_f, [idx], ones16)
        return ()

    lax.fori_loop(0, CHT_LAST, body, ())

    @pl.when(s < NSUB - 1)
    def _():
        lax.fori_loop(CHT_LAST, CHT, body, ())

    def tile_rows(r, _):
        for k in range(CHUNK // 16):
            deg_v[r, pl.ds(k * 16, 16)] = deg_f[pl.ds(r * CHUNK + k * 16, 16)]
        return ()

    lax.fori_loop(0, DEG_R, tile_rows, ())
    pltpu.sync_copy(deg_v, acc_sh.at[rows_i], add=True)
    plsc.subcore_barrier()

    @pl.when(s == 0)
    def _():
        pltpu.sync_copy(acc_sh, deg_hbm.at[c])


@functools.partial(
    pl.kernel,
    out_type=jax.ShapeDtypeStruct((G, N_NODES, D), jnp.float32),
    mesh=_sc_mesh,
    scratch_types=[
        pltpu.VMEM((2, 2, KB, CHUNK), jnp.int32),
        pltpu.VMEM((CHUNK, D), jnp.float32),
        pltpu.VMEM((CHUNK, D), jnp.float32),
        pltpu.VMEM_SHARED((N_NODES, D), jnp.float32),
        pltpu.SemaphoreType.DMA,
        pltpu.SemaphoreType.DMA,
        pltpu.SemaphoreType.DMA,
        pltpu.SemaphoreType.DMA,
        pltpu.SemaphoreType.DMA,
    ],
)
def _sc_aggregate(hp_hbm, ei_hbm, acc_hbm, idx4, rows_a, rows_b,
                  acc_sh, gsa, gsb, ssa, ssb, stsem):
    c = lax.axis_index("c")
    s = lax.axis_index("s")
    row0 = s * ROWS_T
    chunk0 = s * CHT

    @pl.when(s < NSUB - 1)
    def _():
        pltpu.sync_copy(hp_hbm.at[pl.ds(c * N_NODES + row0, ROWS_T)],
                        acc_sh.at[pl.ds(row0, ROWS_T)])

    @pl.when(s == NSUB - 1)
    def _():
        pltpu.sync_copy(hp_hbm.at[pl.ds(c * N_NODES + 15 * ROWS_T, ROWS_LAST)],
                        acc_sh.at[pl.ds(15 * ROWS_T, ROWS_LAST)])

    coff = c * N_NODES
    bufs = [(rows_a, gsa, ssa), (rows_b, gsb, ssb)]

    def add_off(par, nrows):
        def rowfn(r, _):
            for jj in range(CHUNK // 16):
                sl = pl.ds(jj * 16, 16)
                idx4[par, 0, r, sl] = idx4[par, 0, r, sl] + coff
            return ()

        lax.fori_loop(0, nrows, rowfn, ())

    def stage_async(par, b):
        pltpu.async_copy(ei_hbm.at[pl.ds(2 * c, 2), pl.ds(chunk0 + b * KB, KB)],
                         idx4.at[par], stsem)

    def wait_stage(par):
        pltpu.make_async_copy(ei_hbm.at[pl.ds(0, 2), pl.ds(0, KB)],
                              idx4.at[par], stsem).wait()

    def gissue(par, j, p):
        pltpu.async_copy(hp_hbm.at[idx4.at[par, 0, j]], bufs[p][0], bufs[p][1])

    def gwait(par, j, p):
        pltpu.make_async_copy(hp_hbm.at[idx4.at[par, 0, j]],
                              bufs[p][0], bufs[p][1]).wait()

    def proc_batch(par, npar, cross, restage_b):
        for j in range(KB):
            rows, _, ss = bufs[j % 2]
            gwait(par, j, j % 2)
            if j == KB - 2 and cross:
                wait_stage(npar)
                add_off(npar, KB)
            pltpu.async_copy(rows, acc_sh.at[idx4.at[par, 1, j]], ss,
                             add=True).wait()
            if j < KB - 2:
                gissue(par, j + 2, j % 2)
            elif cross:
                gissue(npar, j + 2 - KB, j % 2)
        if restage_b is not None:
            stage_async(par, restage_b)

    pltpu.sync_copy(ei_hbm.at[pl.ds(2 * c, 2), pl.ds(chunk0, KB)],
                    idx4.at[0])
    add_off(0, KB)
    stage_async(1, 1)
    gissue(0, 0, 0)
    gissue(0, 1, 1)
    plsc.subcore_barrier()

    nb = jnp.where(s < NSUB - 1, NB, NB_LAST)

    def body(i, _):
        par = lax.bitwise_and(i, 1)
        proc_batch(par, 1 - par, True, i + 2)
        return ()

    lax.fori_loop(0, nb - 2, body, ())
    proc_batch(0, 1, True, None)
    proc_batch(1, 0, False, None)

    @pl.when(s == NSUB - 1)
    def _():
        off = 15 * CHT + NB_LAST * KB
        pltpu.sync_copy(ei_hbm.at[pl.ds(2 * c, 2), pl.ds(off, TAIL)],
                        idx4.at[0, pl.ds(0, 2), pl.ds(0, TAIL)])
        add_off(0, TAIL)
        for j in range(TAIL):
            pltpu.async_copy(hp_hbm.at[idx4.at[0, 0, j]], rows_a, gsa).wait()
            pltpu.async_copy(rows_a, acc_sh.at[idx4.at[0, 1, j]], ssa,
                             add=True).wait()

    plsc.subcore_barrier()

    @pl.when(s < NSUB - 1)
    def _():
        pltpu.sync_copy(acc_sh.at[pl.ds(row0, ROWS_T)],
                        acc_hbm.at[c, pl.ds(row0, ROWS_T)])

    @pl.when(s == NSUB - 1)
    def _():
        pltpu.sync_copy(acc_sh.at[pl.ds(15 * ROWS_T, ROWS_LAST)],
                        acc_hbm.at[c, pl.ds(15 * ROWS_T, ROWS_LAST)])


def _dinv(deg_ref):
    return lax.rsqrt(deg_ref[0] + 1.0)


def _tc_first_body(deg_ref, x_ref, w_ref, hp_ref):
    h = jnp.dot(x_ref[0], w_ref[0], preferred_element_type=jnp.float32,
                precision=lax.Precision.HIGHEST)
    hp_ref[0] = h * _dinv(deg_ref)


def _tc_mid_body(deg_ref, acc_ref, b_ref, w_ref, hp_ref):
    dinv = _dinv(deg_ref)
    pre = acc_ref[0] * dinv + b_ref[0]
    h = jnp.dot(pre, w_ref[0], preferred_element_type=jnp.float32,
                precision=lax.Precision.HIGHEST)
    hp_ref[0] = h * dinv


def _tc_last_body(deg_ref, acc_ref, b_ref, out_ref):
    out_ref[0] = acc_ref[0] * _dinv(deg_ref) + b_ref[0]


def _bspec(shape):
    return pl.BlockSpec((1,) + shape, lambda g: (g,) + (0,) * len(shape))


_deg_spec = _bspec((N_NODES, D))
_nd_spec = _bspec((N_NODES, D))
_w_spec = _bspec((D, D))
_b_spec = _bspec((1, D))
_nd_out = jax.ShapeDtypeStruct((G, N_NODES, D), jnp.float32)

_tc_first = pl.pallas_call(
    _tc_first_body, grid=(G,),
    in_specs=[_deg_spec, _nd_spec, _w_spec],
    out_specs=_nd_spec, out_shape=_nd_out)

_tc_mid = pl.pallas_call(
    _tc_mid_body, grid=(G,),
    in_specs=[_deg_spec, _nd_spec, _b_spec, _w_spec],
    out_specs=_nd_spec, out_shape=_nd_out)

_tc_last = pl.pallas_call(
    _tc_last_body, grid=(G,),
    in_specs=[_deg_spec, _nd_spec, _b_spec],
    out_specs=_nd_spec, out_shape=_nd_out)


def kernel(x0, edge_index0, x1, edge_index1,
           W0_0, b0_0, W0_1, b0_1, W0_2, b0_2,
           W1_0, b1_0, W1_1, b1_1, W1_2, b1_2):
    ei = jnp.stack([edge_index0.astype(jnp.int32),
                    edge_index1.astype(jnp.int32)])
    eiR = ei.reshape(2 * G, NCH, CHUNK)
    X = jnp.stack([x0, x1])
    Ws = [jnp.stack([W0_0, W1_0]), jnp.stack([W0_1, W1_1]),
          jnp.stack([W0_2, W1_2])]
    Bs = [jnp.stack([b0_0, b1_0]).reshape(G, 1, D),
          jnp.stack([b0_1, b1_1]).reshape(G, 1, D),
          jnp.stack([b0_2, b1_2]).reshape(G, 1, D)]

    deg_flat = _sc_degree(eiR).reshape(G, DEG_R * CHUNK)[:, :N_NODES]
    deg16 = jnp.broadcast_to(deg_flat[:, :, None], (G, N_NODES, D))
    hp = _tc_first(deg16, X, Ws[0])
    acc = _sc_aggregate(hp.reshape(G * N_NODES, D), eiR)
    hp = _tc_mid(deg16, acc, Bs[0], Ws[1])
    acc = _sc_aggregate(hp.reshape(G * N_NODES, D), eiR)
    hp = _tc_mid(deg16, acc, Bs[1], Ws[2])
    acc = _sc_aggregate(hp.reshape(G * N_NODES, D), eiR)
    out = _tc_last(deg16, acc, Bs[2])
    return out.reshape(G * N_NODES, D)
